# baseline (speedup 1.0000x reference)
import jax
import jax.numpy as jnp
from jax.experimental import pallas as pl
from jax.experimental.pallas import tpu as pltpu

_B = 1024
_F = 64
_N_LUT = 100000
_N_CQ = 5000
_N = _N_LUT + _N_CQ
_SCALAR = 30.0
_NPART = 7
_IGNORE = 5555
_NB = 2048
_GRID = (_N + _NB - 1) // _NB
_FULL_LUT = _N_LUT // _NB
_SEAM = _FULL_LUT * _NB
_AUG = (_GRID - _FULL_LUT) * _NB
_TAIL_VALID = _N - (_GRID - 1) * _NB
_NEG = float("-inf")


def _oim_block(xt_ref, lutt_ref, augt_ref, trt_ref, w_ref, logits_ref,
               loss_ref, m_ref, s_ref):
    i = pl.program_id(0)

    def _dot_store(tab_t, xt):
        h = _NB // 2
        logits_ref[pl.ds(0, h), :] = jax.lax.dot_general(
            tab_t[:, :h], xt, (((0,), (0,)), ((), ())),
            preferred_element_type=jnp.float32) * _SCALAR
        logits_ref[pl.ds(h, h), :] = jax.lax.dot_general(
            tab_t[:, h:], xt, (((0,), (0,)), ((), ())),
            preferred_element_type=jnp.float32) * _SCALAR

    def _softmax_step(lm):
        bm = jnp.max(lm, axis=0, keepdims=True)
        bs = jnp.sum(jnp.exp(lm - bm), axis=0, keepdims=True)
        m_old = m_ref[...]
        m_new = jnp.maximum(m_old, bm)
        s_new = (s_ref[...] * jnp.exp(m_old - m_new)
                 + bs * jnp.exp(bm - m_new))
        return m_new, s_new

    @pl.when(i == 0)
    def _init():
        m_ref[...] = jnp.full((1, _B), _NEG, jnp.float32)
        s_ref[...] = jnp.zeros((1, _B), jnp.float32)

    @pl.when(i < _FULL_LUT)
    def _lut_block():
        xt = xt_ref[...]
        _dot_store(
            lutt_ref[:, pl.ds(jnp.minimum(i, _FULL_LUT - 1) * _NB, _NB)], xt)
        m_new, s_new = _softmax_step(logits_ref[...])
        m_ref[...] = m_new
        s_ref[...] = s_new

    @pl.when((i >= _FULL_LUT) & (i < _GRID - 1))
    def _aug_block():
        xt = xt_ref[...]
        j = jnp.clip(i - _FULL_LUT, 0, _AUG // _NB - 1)
        _dot_store(augt_ref[:, pl.ds(j * _NB, _NB)], xt)
        m_new, s_new = _softmax_step(logits_ref[...])
        m_ref[...] = m_new
        s_ref[...] = s_new

    @pl.when(i == _GRID - 1)
    def _tail_block_and_finish():
        xt = xt_ref[...]
        _dot_store(augt_ref[:, pl.ds(_AUG - _NB, _NB)], xt)
        rid = jax.lax.broadcasted_iota(jnp.int32, (_NB, 1), 0)
        lm = jnp.where(rid < _TAIL_VALID, logits_ref[...],
                       jnp.float32(_NEG))
        m_new, s_new = _softmax_step(lm)
        lse = m_new + jnp.log(s_new)
        tl = jnp.sum(xt * trt_ref[...], axis=0, keepdims=True) * _SCALAR
        nll = lse - tl
        w = w_ref[...]
        num = jnp.sum(w * nll, axis=1, keepdims=True)
        den = jnp.maximum(jnp.sum(w, axis=1, keepdims=True), 1e-12)
        loss_ref[...] = num / den


def kernel(inputs, targets, pad_ratios_bidirection, part_idx, lut, cq, weight):
    uppad_ratios = pad_ratios_bidirection[:, 0]
    pad_ratios = pad_ratios_bidirection[:, 1]
    vis_part_up = _NPART - jnp.ceil(_NPART * (1.0 - uppad_ratios))
    vis_part_down = jnp.ceil(_NPART * (1.0 - pad_ratios))
    invis = (part_idx > vis_part_down) | (part_idx <= vis_part_up)
    unlab = targets < 0
    new_targets = jnp.where(invis | unlab, _IGNORE, targets)
    valid = new_targets != _IGNORE
    safe_t = jnp.where(valid, new_targets, 0)
    w = weight[safe_t] * valid.astype(jnp.float32)

    xt = inputs.T
    lut_t = lut.T
    aug_t = jnp.concatenate(
        [lut[_SEAM:], cq,
         jnp.zeros((_AUG - (_N - _SEAM), _F), jnp.float32)], axis=0).T
    trt = jnp.take(lut_t, safe_t, axis=1)
    logits_t, loss = pl.pallas_call(
        _oim_block,
        grid=(_GRID,),
        in_specs=[
            pl.BlockSpec((_F, _B), lambda i: (0, 0)),
            pl.BlockSpec((_F, _N_LUT), lambda i: (0, 0)),
            pl.BlockSpec((_F, _AUG), lambda i: (0, 0)),
            pl.BlockSpec((_F, _B), lambda i: (0, 0)),
            pl.BlockSpec((1, _B), lambda i: (0, 0)),
        ],
        out_specs=[
            pl.BlockSpec((_NB, _B), lambda i: (i, 0)),
            pl.BlockSpec((1, 1), lambda i: (0, 0)),
        ],
        out_shape=[
            jax.ShapeDtypeStruct((_N, _B), jnp.float32),
            jax.ShapeDtypeStruct((1, 1), jnp.float32),
        ],
        scratch_shapes=[pltpu.VMEM((1, _B), jnp.float32)] * 2,
        compiler_params=pltpu.CompilerParams(
            vmem_limit_bytes=63 * 1024 * 1024),
    )(xt, lut_t, aug_t, trt, w.reshape(1, _B))
    return loss[0, 0], jnp.transpose(logits_t)


# speedup vs baseline: 1.0140x; 1.0140x over previous
import jax
import jax.numpy as jnp
from jax.experimental import pallas as pl
from jax.experimental.pallas import tpu as pltpu

_B = 1024
_F = 64
_N_LUT = 100000
_N_CQ = 5000
_N = _N_LUT + _N_CQ
_SCALAR = 30.0
_NPART = 7
_IGNORE = 5555
_NB = 2048
_GRID = (_N + _NB - 1) // _NB
_FULL_LUT = _N_LUT // _NB
_SEAM = _FULL_LUT * _NB
_AUG = (_GRID - _FULL_LUT) * _NB
_TAIL_VALID = _N - (_GRID - 1) * _NB
_NEG = float("-inf")


def _oim_block(xt_ref, lutt_ref, augt_ref, trt_ref, w_ref, logits_ref,
               loss_ref, m_ref, s_ref):
    i = pl.program_id(0)

    def _dot_store(tab_t, xt):
        logits_ref[...] = jax.lax.dot_general(
            tab_t, xt, (((0,), (0,)), ((), ())),
            preferred_element_type=jnp.float32) * _SCALAR

    def _softmax_step(lm):
        bm = jnp.max(lm, axis=0, keepdims=True)
        bs = jnp.sum(jnp.exp(lm - bm), axis=0, keepdims=True)
        m_old = m_ref[...]
        m_new = jnp.maximum(m_old, bm)
        s_new = (s_ref[...] * jnp.exp(m_old - m_new)
                 + bs * jnp.exp(bm - m_new))
        return m_new, s_new

    @pl.when(i == 0)
    def _init():
        m_ref[...] = jnp.full((1, _B), _NEG, jnp.float32)
        s_ref[...] = jnp.zeros((1, _B), jnp.float32)

    @pl.when(i < _FULL_LUT)
    def _lut_block():
        xt = xt_ref[...]
        _dot_store(
            lutt_ref[:, pl.ds(jnp.minimum(i, _FULL_LUT - 1) * _NB, _NB)], xt)
        m_new, s_new = _softmax_step(logits_ref[...])
        m_ref[...] = m_new
        s_ref[...] = s_new

    @pl.when((i >= _FULL_LUT) & (i < _GRID - 1))
    def _aug_block():
        xt = xt_ref[...]
        j = jnp.clip(i - _FULL_LUT, 0, _AUG // _NB - 1)
        _dot_store(augt_ref[:, pl.ds(j * _NB, _NB)], xt)
        m_new, s_new = _softmax_step(logits_ref[...])
        m_ref[...] = m_new
        s_ref[...] = s_new

    @pl.when(i == _GRID - 1)
    def _tail_block_and_finish():
        xt = xt_ref[...]
        _dot_store(augt_ref[:, pl.ds(_AUG - _NB, _NB)], xt)
        rid = jax.lax.broadcasted_iota(jnp.int32, (_NB, 1), 0)
        lm = jnp.where(rid < _TAIL_VALID, logits_ref[...],
                       jnp.float32(_NEG))
        m_new, s_new = _softmax_step(lm)
        lse = m_new + jnp.log(s_new)
        tl = jnp.sum(xt * trt_ref[...], axis=0, keepdims=True) * _SCALAR
        nll = lse - tl
        w = w_ref[...]
        num = jnp.sum(w * nll, axis=1, keepdims=True)
        den = jnp.maximum(jnp.sum(w, axis=1, keepdims=True), 1e-12)
        loss_ref[...] = num / den


def kernel(inputs, targets, pad_ratios_bidirection, part_idx, lut, cq, weight):
    uppad_ratios = pad_ratios_bidirection[:, 0]
    pad_ratios = pad_ratios_bidirection[:, 1]
    vis_part_up = _NPART - jnp.ceil(_NPART * (1.0 - uppad_ratios))
    vis_part_down = jnp.ceil(_NPART * (1.0 - pad_ratios))
    invis = (part_idx > vis_part_down) | (part_idx <= vis_part_up)
    unlab = targets < 0
    new_targets = jnp.where(invis | unlab, _IGNORE, targets)
    valid = new_targets != _IGNORE
    safe_t = jnp.where(valid, new_targets, 0)
    w = weight[safe_t] * valid.astype(jnp.float32)

    xt = inputs.T
    lut_t = lut.T
    aug_t = jnp.concatenate(
        [lut[_SEAM:], cq,
         jnp.zeros((_AUG - (_N - _SEAM), _F), jnp.float32)], axis=0).T
    trt = jnp.take(lut_t, safe_t, axis=1)
    logits_t, loss = pl.pallas_call(
        _oim_block,
        grid=(_GRID,),
        in_specs=[
            pl.BlockSpec((_F, _B), lambda i: (0, 0)),
            pl.BlockSpec((_F, _N_LUT), lambda i: (0, 0)),
            pl.BlockSpec((_F, _AUG), lambda i: (0, 0)),
            pl.BlockSpec((_F, _B), lambda i: (0, 0)),
            pl.BlockSpec((1, _B), lambda i: (0, 0)),
        ],
        out_specs=[
            pl.BlockSpec((_NB, _B), lambda i: (i, 0)),
            pl.BlockSpec((1, 1), lambda i: (0, 0)),
        ],
        out_shape=[
            jax.ShapeDtypeStruct((_N, _B), jnp.float32),
            jax.ShapeDtypeStruct((1, 1), jnp.float32),
        ],
        scratch_shapes=[pltpu.VMEM((1, _B), jnp.float32)] * 2,
        compiler_params=pltpu.CompilerParams(
            vmem_limit_bytes=63 * 1024 * 1024),
    )(xt, lut_t, aug_t, trt, w.reshape(1, _B))
    return loss[0, 0], jnp.transpose(logits_t)
